# baseline (speedup 1.0000x reference)
import jax
import jax.numpy as jnp
from jax import lax
from jax.experimental import pallas as pl
from jax.experimental.pallas import tpu as pltpu
from jax.experimental.pallas import tpu_sc as plsc

D = 64
B = 16384

NW = 32
BPW = B // NW
HALF = BPW // 2
NG = HALF // 16

_MESH = dict(core_axis_name="c", subcore_axis_name="s")


def _row_copies_3d(tbl3_hbm, rows_v, idx_v, hb, g, sem):
    i16 = idx_v[pl.ds(hb + g * 16, 16)]
    return [
        pltpu.make_async_copy(
            tbl3_hbm.at[i16[l] >> 3, pl.ds(i16[l] & 7, 1), :],
            rows_v.at[pl.ds(g * 16 + l, 1), :], sem)
        for l in range(16)
    ]


def _row_copies_2d(tbl_hbm, rows_v, idx_v, hb, g, sem):
    i16 = idx_v[pl.ds(hb + g * 16, 16)]
    return [
        pltpu.make_async_copy(
            tbl_hbm.at[pl.ds(i16[l], 1), :],
            rows_v.at[pl.ds(g * 16 + l, 1), :], sem)
        for l in range(16)
    ]


def _gather_u_kernel(uid_hbm, ut3_hbm, urows_hbm, uidx_v, rows_v, sem):
    wid = lax.axis_index("s") * 2 + lax.axis_index("c")
    base = wid * BPW
    pltpu.sync_copy(uid_hbm.at[pl.ds(base, BPW)], uidx_v)

    for half in range(2):
        hb = half * HALF

        def fire(g, _):
            for c in _row_copies_3d(ut3_hbm, rows_v, uidx_v, hb, g, sem):
                c.start()
            return _

        lax.fori_loop(0, NG, fire, None)

        def drain(g, _):
            for c in _row_copies_3d(ut3_hbm, rows_v, uidx_v, hb, g, sem):
                c.wait()
            return _

        lax.fori_loop(0, NG, drain, None)

        pltpu.sync_copy(rows_v, urows_hbm.at[pl.ds(base + hb, HALF), :])


def _dot_kernel(aid_hbm, at_hbm, urows_hbm, out_hbm,
                aidx_v, urows_v, arows_v, out_v, sem, usem):
    wid = lax.axis_index("s") * 2 + lax.axis_index("c")
    base = wid * BPW
    pltpu.sync_copy(aid_hbm.at[pl.ds(base, BPW)], aidx_v)

    lane = lax.iota(jnp.int32, 16)

    for half in range(2):
        hb = half * HALF

        ucopy = pltpu.make_async_copy(
            urows_hbm.at[pl.ds(base + hb, HALF), :], urows_v, usem)
        ucopy.start()

        def fire(g, _):
            for c in _row_copies_2d(at_hbm, arows_v, aidx_v, hb, g, sem):
                c.start()
            return _

        lax.fori_loop(0, NG, fire, None)

        def drain(g, _):
            for c in _row_copies_2d(at_hbm, arows_v, aidx_v, hb, g, sem):
                c.wait()
            return _

        lax.fori_loop(0, NG, drain, None)
        ucopy.wait()

        def group_body(g, _):
            rv = g * 16 + lane
            acc = jnp.zeros((16,), jnp.float32)
            for d in range(D):
                dv = (jnp.full((16,), d, jnp.int32) + lane) & (D - 1)
                uu = plsc.load_gather(urows_v, [rv, dv])
                aa = plsc.load_gather(arows_v, [rv, dv])
                acc = acc + uu * aa
            out_v[pl.ds(hb + g * 16, 16)] = 1.0 / (1.0 + jnp.exp(-acc))
            return _

        lax.fori_loop(0, NG, group_body, None)

    pltpu.sync_copy(out_v, out_hbm.at[pl.ds(base, BPW)])


@jax.jit
def kernel(user_ids, anime_ids, user_table, anime_table):
    cp = pltpu.CompilerParams(needs_layout_passes=False)

    gather_u = pl.kernel(
        _gather_u_kernel,
        out_type=jax.ShapeDtypeStruct((B, D), jnp.float32),
        mesh=plsc.VectorSubcoreMesh(**_MESH),
        compiler_params=cp,
        scratch_types=[
            pltpu.VMEM((BPW,), jnp.int32),
            pltpu.VMEM((HALF, D), jnp.float32),
            pltpu.SemaphoreType.DMA,
        ],
    )

    dot = pl.kernel(
        _dot_kernel,
        out_type=jax.ShapeDtypeStruct((B,), jnp.float32),
        mesh=plsc.VectorSubcoreMesh(**_MESH),
        compiler_params=cp,
        scratch_types=[
            pltpu.VMEM((BPW,), jnp.int32),
            pltpu.VMEM((HALF, D), jnp.float32),
            pltpu.VMEM((HALF, D), jnp.float32),
            pltpu.VMEM((BPW,), jnp.float32),
            pltpu.SemaphoreType.DMA,
            pltpu.SemaphoreType.DMA,
        ],
    )

    urows = gather_u(user_ids.astype(jnp.int32),
                     user_table.reshape(12500, 8, D))
    return dot(anime_ids.astype(jnp.int32), anime_table, urows)


# speedup vs baseline: 1.1281x; 1.1281x over previous
import jax
import jax.numpy as jnp
from jax import lax
from jax.experimental import pallas as pl
from jax.experimental.pallas import tpu as pltpu
from jax.experimental.pallas import tpu_sc as plsc

D = 64
B = 16384

NW = 32
BPW = B // NW
HALF = BPW // 2
NG = HALF // 16


def _row_copies(ut3_hbm, at3_hbm, urows_v, arows_v, uidx_v, aidx_v, hb, g, sem):
    u16 = uidx_v[pl.ds(hb + g * 16, 16)]
    a16 = aidx_v[pl.ds(hb + g * 16, 16)]
    copies = []
    for l in range(16):
        copies.append(pltpu.make_async_copy(
            ut3_hbm.at[u16[l] >> 3, pl.ds(u16[l] & 7, 1), :],
            urows_v.at[pl.ds(g * 16 + l, 1), :], sem))
        copies.append(pltpu.make_async_copy(
            at3_hbm.at[a16[l] >> 3, pl.ds(a16[l] & 7, 1), :],
            arows_v.at[pl.ds(g * 16 + l, 1), :], sem))
    return copies


def _sc_kernel(uid_hbm, aid_hbm, ut3_hbm, at3_hbm, out_hbm,
               uidx_v, aidx_v, urows_v, arows_v, out_v, sem):
    wid = lax.axis_index("s") * 2 + lax.axis_index("c")
    base = wid * BPW

    pltpu.sync_copy(uid_hbm.at[pl.ds(base, BPW)], uidx_v)
    pltpu.sync_copy(aid_hbm.at[pl.ds(base, BPW)], aidx_v)

    lane = lax.iota(jnp.int32, 16)

    for half in range(2):
        hb = half * HALF

        def fire(g, _):
            for c in _row_copies(ut3_hbm, at3_hbm, urows_v, arows_v,
                                 uidx_v, aidx_v, hb, g, sem):
                c.start()
            return _

        lax.fori_loop(0, NG, fire, None)

        def drain(g, _):
            for c in _row_copies(ut3_hbm, at3_hbm, urows_v, arows_v,
                                 uidx_v, aidx_v, hb, g, sem):
                c.wait()
            return _

        lax.fori_loop(0, NG, drain, None)

        def group_body(g, _):
            rv = g * 16 + lane
            acc = jnp.zeros((16,), jnp.float32)
            for d in range(D):
                dv = (jnp.full((16,), d, jnp.int32) + lane) & (D - 1)
                uu = plsc.load_gather(urows_v, [rv, dv])
                aa = plsc.load_gather(arows_v, [rv, dv])
                acc = acc + uu * aa
            out_v[pl.ds(hb + g * 16, 16)] = 1.0 / (1.0 + jnp.exp(-acc))
            return _

        lax.fori_loop(0, NG, group_body, None)

    pltpu.sync_copy(out_v, out_hbm.at[pl.ds(base, BPW)])


@jax.jit
def kernel(user_ids, anime_ids, user_table, anime_table):
    mesh = plsc.VectorSubcoreMesh(core_axis_name="c", subcore_axis_name="s")
    run = pl.kernel(
        _sc_kernel,
        out_type=jax.ShapeDtypeStruct((B,), jnp.float32),
        mesh=mesh,
        compiler_params=pltpu.CompilerParams(needs_layout_passes=False),
        scratch_types=[
            pltpu.VMEM((BPW,), jnp.int32),
            pltpu.VMEM((BPW,), jnp.int32),
            pltpu.VMEM((HALF, D), jnp.float32),
            pltpu.VMEM((HALF, D), jnp.float32),
            pltpu.VMEM((BPW,), jnp.float32),
            pltpu.SemaphoreType.DMA,
        ],
    )
    return run(user_ids.astype(jnp.int32), anime_ids.astype(jnp.int32),
               user_table.reshape(12500, 8, D), anime_table.reshape(12500, 8, D))
